# baseline (speedup 1.0000x reference)
import functools

import jax
import jax.numpy as jnp
from jax import lax
from jax.experimental import pallas as pl
from jax.experimental.pallas import tpu as pltpu
from jax.experimental.pallas import tpu_sc as plsc

_N = 10000
_NP = 10240
_E = 320000
_H = 128
_G = 256
_W = 64
_NWIN = 160
_NWORK = 32
_EP = _W * _NWIN * _NWORK
_RPS = _NP // 16
_FDIMS = (119, 4, 12, 12, 10, 6, 6, 2, 2)
_FTOT = sum(_FDIMS)

@functools.cache
def _mesh():
    return plsc.VectorSubcoreMesh(core_axis_name="core",
                                  subcore_axis_name="subcore")


_NCH = _NWIN // 4


def _sc_conv_kernel(ms_hbm, src_hbm, dst_hbm, out_hbm,
                    src_v, dst_v, r0, r1, r2, r3, acc_sh,
                    g0, g1, g2, g3, s0, s1, s2, s3):
    cid = lax.axis_index("core")
    sid = lax.axis_index("subcore")
    wid = cid * 16 + sid
    rows = (r0, r1, r2, r3)
    gsem = (g0, g1, g2, g3)
    ssem = (s0, s1, s2, s3)

    @pl.loop(0, _W)
    def _(r):
        @pl.loop(0, _H, step=16)
        def _(c):
            r0.at[pl.ds(r, 1), pl.ds(c, 16)][...] = jnp.zeros(
                (1, 16), jnp.float32)

    @pl.loop(0, _RPS // _W)
    def _(i):
        pltpu.sync_copy(r0, acc_sh.at[pl.ds(sid * _RPS + i * _W, _W)])

    plsc.subcore_barrier()

    for ch in range(_NWIN // _NCH):
        pltpu.sync_copy(src_hbm.at[wid, pl.ds(ch * _NCH, _NCH)], src_v)
        pltpu.sync_copy(dst_hbm.at[wid, pl.ds(ch * _NCH, _NCH)], dst_v)
        for j in range(4):
            pltpu.async_copy(ms_hbm.at[src_v.at[j]], rows[j], gsem[j])

        @pl.loop(0, _NCH, step=4)
        def _(w):
            for j in range(4):
                pltpu.make_async_copy(ms_hbm.at[src_v.at[w + j]], rows[j],
                                      gsem[j]).wait()
                pltpu.async_copy(rows[j], acc_sh.at[dst_v.at[w + j]],
                                 ssem[j], add=True)
            for j in range(4):
                pltpu.make_async_copy(rows[j], acc_sh.at[dst_v.at[w + j]],
                                      ssem[j]).wait()

                @pl.when(w + 4 + j < _NCH)
                def _():
                    pltpu.async_copy(ms_hbm.at[src_v.at[w + 4 + j]], rows[j],
                                     gsem[j])

    plsc.subcore_barrier()
    pltpu.sync_copy(acc_sh.at[pl.ds(sid * _RPS, _RPS)],
                    out_hbm.at[cid, pl.ds(sid * _RPS, _RPS)])


def _sc_conv(ms_pad, src_w, dst_w):
    return pl.kernel(
        _sc_conv_kernel,
        out_type=jax.ShapeDtypeStruct((2, _NP, _H), jnp.float32),
        mesh=_mesh(),
        scratch_types=[
            pltpu.VMEM((_NCH, _W), jnp.int32),
            pltpu.VMEM((_NCH, _W), jnp.int32),
            pltpu.VMEM((_W, _H), jnp.float32),
            pltpu.VMEM((_W, _H), jnp.float32),
            pltpu.VMEM((_W, _H), jnp.float32),
            pltpu.VMEM((_W, _H), jnp.float32),
            pltpu.VMEM_SHARED((_NP, _H), jnp.float32),
            pltpu.SemaphoreType.DMA,
            pltpu.SemaphoreType.DMA,
            pltpu.SemaphoreType.DMA,
            pltpu.SemaphoreType.DMA,
            pltpu.SemaphoreType.DMA,
            pltpu.SemaphoreType.DMA,
            pltpu.SemaphoreType.DMA,
            pltpu.SemaphoreType.DMA,
        ],
    )(ms_pad, src_w, dst_w)


def _sc_deg_kernel(dst_hbm, out_hbm, dst_v, ones_v, acc_sh):
    cid = lax.axis_index("core")
    sid = lax.axis_index("subcore")
    wid = cid * 16 + sid

    @pl.loop(0, _W)
    def _(r):
        ones_v.at[pl.ds(r, 1), pl.ds(0, 16)][...] = jnp.ones((1, 16),
                                                             jnp.float32)

    @pl.loop(0, _RPS // _W)
    def _(i):
        pltpu.sync_copy(ones_v, acc_sh.at[pl.ds(sid * _RPS + i * _W, _W)])

    plsc.subcore_barrier()

    pltpu.sync_copy(dst_hbm.at[wid], dst_v)

    @pl.loop(0, _NWIN)
    def _(w):
        pltpu.sync_copy(ones_v, acc_sh.at[dst_v.at[w]], add=True)

    plsc.subcore_barrier()
    pltpu.sync_copy(acc_sh.at[pl.ds(sid * _RPS, _RPS)],
                    out_hbm.at[cid, pl.ds(sid * _RPS, _RPS)])


def _sc_deg(dst_w):
    return pl.kernel(
        _sc_deg_kernel,
        out_type=jax.ShapeDtypeStruct((2, _NP, 16), jnp.float32),
        mesh=_mesh(),
        scratch_types=[
            pltpu.VMEM((_NWIN, _W), jnp.int32),
            pltpu.VMEM((_W, 16), jnp.float32),
            pltpu.VMEM_SHARED((_NP, 16), jnp.float32),
        ],
    )(dst_w)


def _emb_pre_kernel(x_ref, t_ref, degp_ref, w_ref, o_ref, dinv_ref):
    cols = lax.broadcasted_iota(jnp.int32, (_N, _FTOT), 1)
    mh = jnp.zeros((_N, _FTOT), jnp.float32)
    off = 0
    for f, d in enumerate(_FDIMS):
        mh = mh + (cols == x_ref[:, f:f + 1] + off).astype(jnp.float32)
        off += d
    emb = jnp.dot(mh, t_ref[...], preferred_element_type=jnp.float32)
    deg = degp_ref[0, :_N, 0:1] + degp_ref[1, :_N, 0:1] - 1.0
    dinv = lax.rsqrt(deg)
    dinv_ref[...] = dinv
    ms = jnp.dot(emb, w_ref[...], preferred_element_type=jnp.float32) * dinv
    o_ref[:_N, :] = ms
    o_ref[_N:, :] = jnp.zeros((_NP - _N, _H), jnp.float32)


def _emb_pre(x, tables_cat, deg_parts, W1):
    return pl.pallas_call(
        _emb_pre_kernel,
        out_shape=(jax.ShapeDtypeStruct((_NP, _H), jnp.float32),
                   jax.ShapeDtypeStruct((_N, 1), jnp.float32)),
    )(x, tables_cat, deg_parts, W1.T)


def _post_kernel(p_ref, ms_ref, dinv_ref, b_ref, g_ref, bt_ref, w_ref, o_ref):
    dinv = dinv_ref[...]
    s = p_ref[0, :_N, :] + p_ref[1, :_N, :] + ms_ref[:_N, :]
    h = dinv * s + b_ref[...]
    h = jnp.maximum(h, 0.0)
    mu = jnp.mean(h, axis=0, keepdims=True)
    var = jnp.mean((h - mu) ** 2, axis=0, keepdims=True)
    h = (h - mu) / jnp.sqrt(var + 1e-5) * g_ref[...] + bt_ref[...]
    ms = jnp.dot(h, w_ref[...], preferred_element_type=jnp.float32) * dinv
    o_ref[:_N, :] = ms
    o_ref[_N:, :] = jnp.zeros((_NP - _N, _H), jnp.float32)


def _post(parts, ms_pad, dinv, b, gamma, beta, W_next):
    return pl.pallas_call(
        _post_kernel,
        out_shape=jax.ShapeDtypeStruct((_NP, _H), jnp.float32),
    )(parts, ms_pad, dinv, b[None, :], gamma[None, :], beta[None, :], W_next.T)


def _out3_kernel(p_ref, ms_ref, dinv_ref, b_ref, batch_ref, o_ref, oh_ref):
    s = p_ref[0, :_N, :] + p_ref[1, :_N, :] + ms_ref[:_N, :]
    o_ref[...] = dinv_ref[...] * s + b_ref[...]
    oh_ref[...] = (batch_ref[...] == lax.broadcasted_iota(
        jnp.int32, (_N, _G), 1)).astype(jnp.float32)


def _s2s_kernel(oh_ref, out_ref,
                wih_ref, whh_ref, bih_ref, bhh_ref,
                l1w_ref, l1b_ref, l2w_ref, l2b_ref, z_ref):
    out = out_ref[...]
    onehot = oh_ref[...]

    h = jnp.zeros((_G, _H), jnp.float32)
    c = jnp.zeros((_G, _H), jnp.float32)
    q_star = jnp.zeros((_G, 2 * _H), jnp.float32)
    for _ in range(4):
        g = (jnp.dot(q_star, wih_ref[...],
                     preferred_element_type=jnp.float32) + bih_ref[...]
             + jnp.dot(h, whh_ref[...],
                       preferred_element_type=jnp.float32) + bhh_ref[...])
        i_g, f_g, g_g, o_g = jnp.split(g, 4, axis=-1)
        c = jax.nn.sigmoid(f_g) * c + jax.nn.sigmoid(i_g) * jnp.tanh(g_g)
        h = jax.nn.sigmoid(o_g) * jnp.tanh(c)
        qb = jnp.dot(onehot, h, preferred_element_type=jnp.float32)
        e = jnp.sum(out * qb, axis=1, keepdims=True)
        emax = jnp.max(jnp.where(onehot > 0.0, e, -1e30), axis=0,
                       keepdims=True)
        emax = jnp.where(emax > -1e29, emax, 0.0)
        ee = jnp.exp(e - jnp.sum(onehot * emax, axis=1, keepdims=True))
        den = jnp.dot(ee.T, onehot, preferred_element_type=jnp.float32)
        denb = jnp.sum(onehot * den, axis=1, keepdims=True)
        a = ee / (denb + 1e-16)
        r = lax.dot_general(onehot, a * out, (((0,), (0,)), ((), ())),
                            preferred_element_type=jnp.float32)
        q_star = jnp.concatenate([h, r], axis=1)
    z = jnp.dot(q_star, l1w_ref[...],
                preferred_element_type=jnp.float32) + l1b_ref[...]
    z = jnp.dot(z, l2w_ref[...],
                preferred_element_type=jnp.float32) + l2b_ref[...]
    z_ref[...] = jax.nn.sigmoid(z)


def _final(parts, ms_pad, dinv, b, batch, Wih, Whh, bih, bhh,
           lin1_W, lin1_b, lin2_W, lin2_b):
    out3, onehot = pl.pallas_call(
        _out3_kernel,
        out_shape=(jax.ShapeDtypeStruct((_N, _H), jnp.float32),
                   jax.ShapeDtypeStruct((_N, _G), jnp.float32)),
    )(parts, ms_pad, dinv, b[None, :], batch[:, None].astype(jnp.int32))
    return pl.pallas_call(
        _s2s_kernel,
        out_shape=jax.ShapeDtypeStruct((_G, 1), jnp.float32),
    )(onehot, out3,
      Wih.T, Whh.T, bih[None, :], bhh[None, :],
      lin1_W.T, lin1_b[None, :], lin2_W.T, lin2_b[None, :])


def kernel(x, edge_index, edge_attr, batch, emb_tables, W1, b1, W2, b2, W3, b3,
           bn_gamma, bn_beta, Wih, Whh, bih, bhh, lin1_W, lin1_b, lin2_W, lin2_b):
    del edge_attr
    src = edge_index[0].astype(jnp.int32)
    dst = edge_index[1].astype(jnp.int32)
    pad = _N + (jnp.arange(_EP - _E, dtype=jnp.int32) % (_NP - _N))
    src_w = jnp.concatenate([src, pad]).reshape(_NWORK, _NWIN, _W)
    dst_w = jnp.concatenate([dst, pad]).reshape(_NWORK, _NWIN, _W)

    deg_parts = _sc_deg(dst_w)
    ms1, dinv = _emb_pre(x.astype(jnp.int32),
                         jnp.concatenate(emb_tables, axis=0), deg_parts, W1)

    parts1 = _sc_conv(ms1, src_w, dst_w)
    ms2 = _post(parts1, ms1, dinv, b1, bn_gamma, bn_beta, W2)
    parts2 = _sc_conv(ms2, src_w, dst_w)
    ms3 = _post(parts2, ms2, dinv, b2, bn_gamma, bn_beta, W3)
    parts3 = _sc_conv(ms3, src_w, dst_w)

    return _final(parts3, ms3, dinv, b3, batch, Wih, Whh, bih, bhh,
                  lin1_W, lin1_b, lin2_W, lin2_b)


# speedup vs baseline: 1.0795x; 1.0795x over previous
import functools

import jax
import jax.numpy as jnp
from jax import lax
from jax.experimental import pallas as pl
from jax.experimental.pallas import tpu as pltpu
from jax.experimental.pallas import tpu_sc as plsc

_N = 10000
_NP = 10240
_E = 320000
_H = 128
_G = 256
_W = 128
_NWIN = 80
_NWORK = 32
_EP = _W * _NWIN * _NWORK
_RPS = _NP // 16
_FDIMS = (119, 4, 12, 12, 10, 6, 6, 2, 2)
_FTOT = sum(_FDIMS)

@functools.cache
def _mesh():
    return plsc.VectorSubcoreMesh(core_axis_name="core",
                                  subcore_axis_name="subcore")


_NCH = _NWIN // 2


def _sc_conv_kernel(ms_hbm, src_hbm, dst_hbm, out_hbm,
                    src_v, dst_v, rows0_v, rows1_v, acc_sh, sem0, sem1):
    cid = lax.axis_index("core")
    sid = lax.axis_index("subcore")
    wid = cid * 16 + sid

    @pl.loop(0, _W)
    def _(r):
        @pl.loop(0, _H, step=16)
        def _(c):
            rows0_v.at[pl.ds(r, 1), pl.ds(c, 16)][...] = jnp.zeros(
                (1, 16), jnp.float32)

    @pl.loop(0, _RPS // _W)
    def _(i):
        pltpu.sync_copy(rows0_v, acc_sh.at[pl.ds(sid * _RPS + i * _W, _W)])

    plsc.subcore_barrier()

    for ch in range(_NWIN // _NCH):
        pltpu.sync_copy(src_hbm.at[wid, pl.ds(ch * _NCH, _NCH)], src_v)
        pltpu.sync_copy(dst_hbm.at[wid, pl.ds(ch * _NCH, _NCH)], dst_v)
        pltpu.async_copy(ms_hbm.at[src_v.at[0]], rows0_v, sem0)

        @pl.loop(0, _NCH, step=2)
        def _(w):
            pltpu.async_copy(ms_hbm.at[src_v.at[w + 1]], rows1_v, sem1)
            pltpu.make_async_copy(ms_hbm.at[src_v.at[w]], rows0_v,
                                  sem0).wait()
            pltpu.sync_copy(rows0_v, acc_sh.at[dst_v.at[w]], add=True)

            @pl.when(w + 2 < _NCH)
            def _():
                pltpu.async_copy(ms_hbm.at[src_v.at[w + 2]], rows0_v, sem0)

            pltpu.make_async_copy(ms_hbm.at[src_v.at[w + 1]], rows1_v,
                                  sem1).wait()
            pltpu.sync_copy(rows1_v, acc_sh.at[dst_v.at[w + 1]], add=True)

    plsc.subcore_barrier()
    pltpu.sync_copy(acc_sh.at[pl.ds(sid * _RPS, _RPS)],
                    out_hbm.at[cid, pl.ds(sid * _RPS, _RPS)])


def _sc_conv(ms_pad, src_w, dst_w):
    return pl.kernel(
        _sc_conv_kernel,
        out_type=jax.ShapeDtypeStruct((2, _NP, _H), jnp.float32),
        mesh=_mesh(),
        scratch_types=[
            pltpu.VMEM((_NCH, _W), jnp.int32),
            pltpu.VMEM((_NCH, _W), jnp.int32),
            pltpu.VMEM((_W, _H), jnp.float32),
            pltpu.VMEM((_W, _H), jnp.float32),
            pltpu.VMEM_SHARED((_NP, _H), jnp.float32),
            pltpu.SemaphoreType.DMA,
            pltpu.SemaphoreType.DMA,
        ],
    )(ms_pad, src_w, dst_w)


def _sc_deg_kernel(dst_hbm, out_hbm, dst_v, ones_v, acc_sh):
    cid = lax.axis_index("core")
    sid = lax.axis_index("subcore")
    wid = cid * 16 + sid

    @pl.loop(0, _W)
    def _(r):
        ones_v.at[pl.ds(r, 1), pl.ds(0, 16)][...] = jnp.ones((1, 16),
                                                             jnp.float32)

    @pl.loop(0, _RPS // _W)
    def _(i):
        pltpu.sync_copy(ones_v, acc_sh.at[pl.ds(sid * _RPS + i * _W, _W)])

    plsc.subcore_barrier()

    pltpu.sync_copy(dst_hbm.at[wid], dst_v)

    @pl.loop(0, _NWIN)
    def _(w):
        pltpu.sync_copy(ones_v, acc_sh.at[dst_v.at[w]], add=True)

    plsc.subcore_barrier()
    pltpu.sync_copy(acc_sh.at[pl.ds(sid * _RPS, _RPS)],
                    out_hbm.at[cid, pl.ds(sid * _RPS, _RPS)])


def _sc_deg(dst_w):
    return pl.kernel(
        _sc_deg_kernel,
        out_type=jax.ShapeDtypeStruct((2, _NP, 16), jnp.float32),
        mesh=_mesh(),
        scratch_types=[
            pltpu.VMEM((_NWIN, _W), jnp.int32),
            pltpu.VMEM((_W, 16), jnp.float32),
            pltpu.VMEM_SHARED((_NP, 16), jnp.float32),
        ],
    )(dst_w)


def _emb_pre_kernel(x_ref, t_ref, degp_ref, w_ref, o_ref, dinv_ref):
    cols = lax.broadcasted_iota(jnp.int32, (_N, _FTOT), 1)
    mh = jnp.zeros((_N, _FTOT), jnp.float32)
    off = 0
    for f, d in enumerate(_FDIMS):
        mh = mh + (cols == x_ref[:, f:f + 1] + off).astype(jnp.float32)
        off += d
    emb = jnp.dot(mh, t_ref[...], preferred_element_type=jnp.float32)
    deg = degp_ref[0, :_N, 0:1] + degp_ref[1, :_N, 0:1] - 1.0
    dinv = lax.rsqrt(deg)
    dinv_ref[...] = dinv
    ms = jnp.dot(emb, w_ref[...], preferred_element_type=jnp.float32) * dinv
    o_ref[:_N, :] = ms
    o_ref[_N:, :] = jnp.zeros((_NP - _N, _H), jnp.float32)


def _emb_pre(x, tables_cat, deg_parts, W1):
    return pl.pallas_call(
        _emb_pre_kernel,
        out_shape=(jax.ShapeDtypeStruct((_NP, _H), jnp.float32),
                   jax.ShapeDtypeStruct((_N, 1), jnp.float32)),
    )(x, tables_cat, deg_parts, W1.T)


def _post_kernel(p_ref, ms_ref, dinv_ref, b_ref, g_ref, bt_ref, w_ref, o_ref):
    dinv = dinv_ref[...]
    s = p_ref[0, :_N, :] + p_ref[1, :_N, :] + ms_ref[:_N, :]
    h = dinv * s + b_ref[...]
    h = jnp.maximum(h, 0.0)
    mu = jnp.mean(h, axis=0, keepdims=True)
    var = jnp.mean((h - mu) ** 2, axis=0, keepdims=True)
    h = (h - mu) / jnp.sqrt(var + 1e-5) * g_ref[...] + bt_ref[...]
    ms = jnp.dot(h, w_ref[...], preferred_element_type=jnp.float32) * dinv
    o_ref[:_N, :] = ms
    o_ref[_N:, :] = jnp.zeros((_NP - _N, _H), jnp.float32)


def _post(parts, ms_pad, dinv, b, gamma, beta, W_next):
    return pl.pallas_call(
        _post_kernel,
        out_shape=jax.ShapeDtypeStruct((_NP, _H), jnp.float32),
    )(parts, ms_pad, dinv, b[None, :], gamma[None, :], beta[None, :], W_next.T)


def _out3_kernel(p_ref, ms_ref, dinv_ref, b_ref, batch_ref, o_ref, oh_ref):
    s = p_ref[0, :_N, :] + p_ref[1, :_N, :] + ms_ref[:_N, :]
    o_ref[...] = dinv_ref[...] * s + b_ref[...]
    oh_ref[...] = (batch_ref[...] == lax.broadcasted_iota(
        jnp.int32, (_N, _G), 1)).astype(jnp.float32)


def _s2s_kernel(oh_ref, out_ref,
                wih_ref, whh_ref, bih_ref, bhh_ref,
                l1w_ref, l1b_ref, l2w_ref, l2b_ref, z_ref):
    out = out_ref[...]
    onehot = oh_ref[...]

    h = jnp.zeros((_G, _H), jnp.float32)
    c = jnp.zeros((_G, _H), jnp.float32)
    q_star = jnp.zeros((_G, 2 * _H), jnp.float32)
    for _ in range(4):
        g = (jnp.dot(q_star, wih_ref[...],
                     preferred_element_type=jnp.float32) + bih_ref[...]
             + jnp.dot(h, whh_ref[...],
                       preferred_element_type=jnp.float32) + bhh_ref[...])
        i_g, f_g, g_g, o_g = jnp.split(g, 4, axis=-1)
        c = jax.nn.sigmoid(f_g) * c + jax.nn.sigmoid(i_g) * jnp.tanh(g_g)
        h = jax.nn.sigmoid(o_g) * jnp.tanh(c)
        qb = jnp.dot(onehot, h, preferred_element_type=jnp.float32)
        e = jnp.sum(out * qb, axis=1, keepdims=True)
        emax = jnp.max(jnp.where(onehot > 0.0, e, -1e30), axis=0,
                       keepdims=True)
        emax = jnp.where(emax > -1e29, emax, 0.0)
        ee = jnp.exp(e - jnp.sum(onehot * emax, axis=1, keepdims=True))
        den = jnp.dot(ee.T, onehot, preferred_element_type=jnp.float32)
        denb = jnp.sum(onehot * den, axis=1, keepdims=True)
        a = ee / (denb + 1e-16)
        r = lax.dot_general(onehot, a * out, (((0,), (0,)), ((), ())),
                            preferred_element_type=jnp.float32)
        q_star = jnp.concatenate([h, r], axis=1)
    z = jnp.dot(q_star, l1w_ref[...],
                preferred_element_type=jnp.float32) + l1b_ref[...]
    z = jnp.dot(z, l2w_ref[...],
                preferred_element_type=jnp.float32) + l2b_ref[...]
    z_ref[...] = jax.nn.sigmoid(z)


def _final(parts, ms_pad, dinv, b, batch, Wih, Whh, bih, bhh,
           lin1_W, lin1_b, lin2_W, lin2_b):
    out3, onehot = pl.pallas_call(
        _out3_kernel,
        out_shape=(jax.ShapeDtypeStruct((_N, _H), jnp.float32),
                   jax.ShapeDtypeStruct((_N, _G), jnp.float32)),
    )(parts, ms_pad, dinv, b[None, :], batch[:, None].astype(jnp.int32))
    return pl.pallas_call(
        _s2s_kernel,
        out_shape=jax.ShapeDtypeStruct((_G, 1), jnp.float32),
    )(onehot, out3,
      Wih.T, Whh.T, bih[None, :], bhh[None, :],
      lin1_W.T, lin1_b[None, :], lin2_W.T, lin2_b[None, :])


def kernel(x, edge_index, edge_attr, batch, emb_tables, W1, b1, W2, b2, W3, b3,
           bn_gamma, bn_beta, Wih, Whh, bih, bhh, lin1_W, lin1_b, lin2_W, lin2_b):
    del edge_attr
    src = edge_index[0].astype(jnp.int32)
    dst = edge_index[1].astype(jnp.int32)
    pad = _N + (jnp.arange(_EP - _E, dtype=jnp.int32) % (_NP - _N))
    src_w = jnp.concatenate([src, pad]).reshape(_NWORK, _NWIN, _W)
    dst_w = jnp.concatenate([dst, pad]).reshape(_NWORK, _NWIN, _W)

    deg_parts = _sc_deg(dst_w)
    ms1, dinv = _emb_pre(x.astype(jnp.int32),
                         jnp.concatenate(emb_tables, axis=0), deg_parts, W1)

    parts1 = _sc_conv(ms1, src_w, dst_w)
    ms2 = _post(parts1, ms1, dinv, b1, bn_gamma, bn_beta, W2)
    parts2 = _sc_conv(ms2, src_w, dst_w)
    ms3 = _post(parts2, ms2, dinv, b2, bn_gamma, bn_beta, W3)
    parts3 = _sc_conv(ms3, src_w, dst_w)

    return _final(parts3, ms3, dinv, b3, batch, Wih, Whh, bih, bhh,
                  lin1_W, lin1_b, lin2_W, lin2_b)


# speedup vs baseline: 1.0896x; 1.0093x over previous
import functools

import jax
import jax.numpy as jnp
from jax import lax
from jax.experimental import pallas as pl
from jax.experimental.pallas import tpu as pltpu
from jax.experimental.pallas import tpu_sc as plsc

_N = 10000
_NP = 10240
_E = 320000
_H = 128
_G = 256
_W = 128
_NWIN = 80
_NWORK = 32
_EP = _W * _NWIN * _NWORK
_RPS = _NP // 16
_FDIMS = (119, 4, 12, 12, 10, 6, 6, 2, 2)
_FTOT = sum(_FDIMS)

@functools.cache
def _mesh():
    return plsc.VectorSubcoreMesh(core_axis_name="core",
                                  subcore_axis_name="subcore")


_NCH = _NWIN // 2


def _sc_conv_kernel(ms_hbm, src_hbm, dst_hbm, out_hbm,
                    src_v, dst_v, rows0_v, rows1_v, acc_sh, sem0, sem1):
    cid = lax.axis_index("core")
    sid = lax.axis_index("subcore")
    wid = cid * 16 + sid

    @pl.loop(0, _W)
    def _(r):
        @pl.loop(0, _H, step=16)
        def _(c):
            rows0_v.at[pl.ds(r, 1), pl.ds(c, 16)][...] = jnp.zeros(
                (1, 16), jnp.float32)

    @pl.loop(0, _RPS // _W)
    def _(i):
        pltpu.sync_copy(rows0_v, acc_sh.at[pl.ds(sid * _RPS + i * _W, _W)])

    plsc.subcore_barrier()

    for ch in range(_NWIN // _NCH):
        pltpu.sync_copy(src_hbm.at[wid, pl.ds(ch * _NCH, _NCH)], src_v)
        pltpu.sync_copy(dst_hbm.at[wid, pl.ds(ch * _NCH, _NCH)], dst_v)
        pltpu.async_copy(ms_hbm.at[src_v.at[0]], rows0_v, sem0)

        @pl.loop(0, _NCH, step=2)
        def _(w):
            pltpu.async_copy(ms_hbm.at[src_v.at[w + 1]], rows1_v, sem1)
            pltpu.make_async_copy(ms_hbm.at[src_v.at[w]], rows0_v,
                                  sem0).wait()
            pltpu.sync_copy(rows0_v, acc_sh.at[dst_v.at[w]], add=True)

            @pl.when(w + 2 < _NCH)
            def _():
                pltpu.async_copy(ms_hbm.at[src_v.at[w + 2]], rows0_v, sem0)

            pltpu.make_async_copy(ms_hbm.at[src_v.at[w + 1]], rows1_v,
                                  sem1).wait()
            pltpu.sync_copy(rows1_v, acc_sh.at[dst_v.at[w + 1]], add=True)

    plsc.subcore_barrier()
    pltpu.sync_copy(acc_sh.at[pl.ds(sid * _RPS, _RPS)],
                    out_hbm.at[cid, pl.ds(sid * _RPS, _RPS)])


def _sc_conv(ms_pad, src_w, dst_w):
    return pl.kernel(
        _sc_conv_kernel,
        out_type=jax.ShapeDtypeStruct((2, _NP, _H), jnp.float32),
        mesh=_mesh(),
        scratch_types=[
            pltpu.VMEM((_NCH, _W), jnp.int32),
            pltpu.VMEM((_NCH, _W), jnp.int32),
            pltpu.VMEM((_W, _H), jnp.float32),
            pltpu.VMEM((_W, _H), jnp.float32),
            pltpu.VMEM_SHARED((_NP, _H), jnp.float32),
            pltpu.SemaphoreType.DMA,
            pltpu.SemaphoreType.DMA,
        ],
    )(ms_pad, src_w, dst_w)


def _sc_deg_kernel(dst_hbm, out_hbm, dst_v, ones_v, acc_sh):
    cid = lax.axis_index("core")
    sid = lax.axis_index("subcore")
    wid = cid * 16 + sid

    @pl.loop(0, _W)
    def _(r):
        ones_v.at[pl.ds(r, 1), pl.ds(0, 16)][...] = jnp.ones((1, 16),
                                                             jnp.float32)

    @pl.loop(0, _RPS // _W)
    def _(i):
        pltpu.sync_copy(ones_v, acc_sh.at[pl.ds(sid * _RPS + i * _W, _W)])

    plsc.subcore_barrier()

    pltpu.sync_copy(dst_hbm.at[wid], dst_v)

    @pl.loop(0, _NWIN)
    def _(w):
        pltpu.sync_copy(ones_v, acc_sh.at[dst_v.at[w]], add=True)

    plsc.subcore_barrier()
    pltpu.sync_copy(acc_sh.at[pl.ds(sid * _RPS, _RPS)],
                    out_hbm.at[cid, pl.ds(sid * _RPS, _RPS)])


def _sc_deg(dst_w):
    return pl.kernel(
        _sc_deg_kernel,
        out_type=jax.ShapeDtypeStruct((2, _NP, 16), jnp.float32),
        mesh=_mesh(),
        scratch_types=[
            pltpu.VMEM((_NWIN, _W), jnp.int32),
            pltpu.VMEM((_W, 16), jnp.float32),
            pltpu.VMEM_SHARED((_NP, 16), jnp.float32),
        ],
    )(dst_w)


def _emb_mm_kernel(x_ref, t_ref, w_ref, o_ref):
    cols = lax.broadcasted_iota(jnp.int32, (_N, _FTOT), 1)
    mh = jnp.zeros((_N, _FTOT), jnp.float32)
    off = 0
    for f, d in enumerate(_FDIMS):
        mh = mh + (cols == x_ref[:, f:f + 1] + off).astype(jnp.float32)
        off += d
    emb = jnp.dot(mh, t_ref[...], preferred_element_type=jnp.float32)
    o_ref[...] = jnp.dot(emb, w_ref[...], preferred_element_type=jnp.float32)


def _scale_kernel(m_ref, degp_ref, o_ref, dinv_ref):
    deg = degp_ref[0, :_N, 0:1] + degp_ref[1, :_N, 0:1] - 1.0
    dinv = lax.rsqrt(deg)
    dinv_ref[...] = dinv
    o_ref[:_N, :] = m_ref[...] * dinv
    o_ref[_N:, :] = jnp.zeros((_NP - _N, _H), jnp.float32)


def _emb_pre(x, tables_cat, deg_parts, W1):
    m1 = pl.pallas_call(
        _emb_mm_kernel,
        out_shape=jax.ShapeDtypeStruct((_N, _H), jnp.float32),
    )(x, tables_cat, W1.T)
    return pl.pallas_call(
        _scale_kernel,
        out_shape=(jax.ShapeDtypeStruct((_NP, _H), jnp.float32),
                   jax.ShapeDtypeStruct((_N, 1), jnp.float32)),
    )(m1, deg_parts)


def _post_kernel(p_ref, ms_ref, dinv_ref, b_ref, g_ref, bt_ref, w_ref, o_ref):
    dinv = dinv_ref[...]
    s = p_ref[0, :_N, :] + p_ref[1, :_N, :] + ms_ref[:_N, :]
    h = dinv * s + b_ref[...]
    h = jnp.maximum(h, 0.0)
    mu = jnp.mean(h, axis=0, keepdims=True)
    var = jnp.mean((h - mu) ** 2, axis=0, keepdims=True)
    h = (h - mu) / jnp.sqrt(var + 1e-5) * g_ref[...] + bt_ref[...]
    ms = jnp.dot(h, w_ref[...], preferred_element_type=jnp.float32) * dinv
    o_ref[:_N, :] = ms
    o_ref[_N:, :] = jnp.zeros((_NP - _N, _H), jnp.float32)


def _post(parts, ms_pad, dinv, b, gamma, beta, W_next):
    return pl.pallas_call(
        _post_kernel,
        out_shape=jax.ShapeDtypeStruct((_NP, _H), jnp.float32),
    )(parts, ms_pad, dinv, b[None, :], gamma[None, :], beta[None, :], W_next.T)


def _out3_kernel(p_ref, ms_ref, dinv_ref, b_ref, batch_ref, o_ref, oh_ref):
    s = p_ref[0, :_N, :] + p_ref[1, :_N, :] + ms_ref[:_N, :]
    o_ref[...] = dinv_ref[...] * s + b_ref[...]
    oh_ref[...] = (batch_ref[...] == lax.broadcasted_iota(
        jnp.int32, (_N, _G), 1)).astype(jnp.float32)


def _s2s_kernel(oh_ref, out_ref,
                wih_ref, whh_ref, bih_ref, bhh_ref,
                l1w_ref, l1b_ref, l2w_ref, l2b_ref, z_ref):
    out = out_ref[...]
    onehot = oh_ref[...]

    h = jnp.zeros((_G, _H), jnp.float32)
    c = jnp.zeros((_G, _H), jnp.float32)
    q_star = jnp.zeros((_G, 2 * _H), jnp.float32)
    for _ in range(4):
        g = (jnp.dot(q_star, wih_ref[...],
                     preferred_element_type=jnp.float32) + bih_ref[...]
             + jnp.dot(h, whh_ref[...],
                       preferred_element_type=jnp.float32) + bhh_ref[...])
        i_g, f_g, g_g, o_g = jnp.split(g, 4, axis=-1)
        c = jax.nn.sigmoid(f_g) * c + jax.nn.sigmoid(i_g) * jnp.tanh(g_g)
        h = jax.nn.sigmoid(o_g) * jnp.tanh(c)
        qb = jnp.dot(onehot, h, preferred_element_type=jnp.float32)
        e = jnp.sum(out * qb, axis=1, keepdims=True)
        emax = jnp.max(jnp.where(onehot > 0.0, e, -1e30), axis=0,
                       keepdims=True)
        emax = jnp.where(emax > -1e29, emax, 0.0)
        ee = jnp.exp(e - jnp.sum(onehot * emax, axis=1, keepdims=True))
        den = jnp.dot(ee.T, onehot, preferred_element_type=jnp.float32)
        denb = jnp.sum(onehot * den, axis=1, keepdims=True)
        a = ee / (denb + 1e-16)
        r = lax.dot_general(onehot, a * out, (((0,), (0,)), ((), ())),
                            preferred_element_type=jnp.float32)
        q_star = jnp.concatenate([h, r], axis=1)
    z = jnp.dot(q_star, l1w_ref[...],
                preferred_element_type=jnp.float32) + l1b_ref[...]
    z = jnp.dot(z, l2w_ref[...],
                preferred_element_type=jnp.float32) + l2b_ref[...]
    z_ref[...] = jax.nn.sigmoid(z)


def _final(parts, ms_pad, dinv, b, batch, Wih, Whh, bih, bhh,
           lin1_W, lin1_b, lin2_W, lin2_b):
    out3, onehot = pl.pallas_call(
        _out3_kernel,
        out_shape=(jax.ShapeDtypeStruct((_N, _H), jnp.float32),
                   jax.ShapeDtypeStruct((_N, _G), jnp.float32)),
    )(parts, ms_pad, dinv, b[None, :], batch[:, None].astype(jnp.int32))
    return pl.pallas_call(
        _s2s_kernel,
        out_shape=jax.ShapeDtypeStruct((_G, 1), jnp.float32),
    )(onehot, out3,
      Wih.T, Whh.T, bih[None, :], bhh[None, :],
      lin1_W.T, lin1_b[None, :], lin2_W.T, lin2_b[None, :])


def kernel(x, edge_index, edge_attr, batch, emb_tables, W1, b1, W2, b2, W3, b3,
           bn_gamma, bn_beta, Wih, Whh, bih, bhh, lin1_W, lin1_b, lin2_W, lin2_b):
    del edge_attr
    src = edge_index[0].astype(jnp.int32)
    dst = edge_index[1].astype(jnp.int32)
    pad = _N + (jnp.arange(_EP - _E, dtype=jnp.int32) % (_NP - _N))
    src_w = jnp.concatenate([src, pad]).reshape(_NWORK, _NWIN, _W)
    dst_w = jnp.concatenate([dst, pad]).reshape(_NWORK, _NWIN, _W)

    deg_parts = _sc_deg(dst_w)
    ms1, dinv = _emb_pre(x.astype(jnp.int32),
                         jnp.concatenate(emb_tables, axis=0), deg_parts, W1)

    parts1 = _sc_conv(ms1, src_w, dst_w)
    ms2 = _post(parts1, ms1, dinv, b1, bn_gamma, bn_beta, W2)
    parts2 = _sc_conv(ms2, src_w, dst_w)
    ms3 = _post(parts2, ms2, dinv, b2, bn_gamma, bn_beta, W3)
    parts3 = _sc_conv(ms3, src_w, dst_w)

    return _final(parts3, ms3, dinv, b3, batch, Wih, Whh, bih, bhh,
                  lin1_W, lin1_b, lin2_W, lin2_b)
